# baseline (speedup 1.0000x reference)
import functools

import jax
import jax.numpy as jnp
from jax import lax
from jax.experimental import pallas as pl
from jax.experimental.pallas import tpu as pltpu
from jax.experimental.pallas import tpu_sc as plsc

_C = 128
_NBUF = 4
_DIST = 2


@functools.lru_cache(maxsize=None)
def _build_sc_gather(n, v, d, dtype_name):
    dtype = jnp.dtype(dtype_name)
    info = plsc.get_sparse_core_info()
    nc, ns = info.num_cores, info.num_subcores
    nw = nc * ns
    full = n // _C
    tail = n % _C
    base = full // nw
    extra = full % nw
    assert base >= _NBUF and tail % 8 == 0 and extra < nw - 1

    len_lo = base * _C
    len_hi = (base + 1) * _C
    len_last = base * _C + tail

    mesh = plsc.VectorSubcoreMesh(core_axis_name="c", subcore_axis_name="s")

    scratch = [
        pltpu.VMEM((len_hi,), jnp.int32),
        pltpu.VMEM((_C, d), dtype),
        pltpu.VMEM((_C, d), dtype),
        pltpu.VMEM((_C, d), dtype),
        pltpu.VMEM((_C, d), dtype),
        pltpu.VMEM((max(tail, 8), d), dtype),
        pltpu.SemaphoreType.DMA((_NBUF,)),
        pltpu.SemaphoreType.DMA((_NBUF,)),
        pltpu.VMEM_SHARED((v, d), dtype),
    ]

    @functools.partial(
        pl.kernel,
        mesh=mesh,
        out_type=jax.ShapeDtypeStruct((n, d), dtype),
        scratch_types=scratch,
    )
    def gather_kernel(idx_hbm, table_hbm, out_hbm, idx_all,
                      r0, r1, r2, r3, rows_t, gsem, ssem, tab_sp):
        rows = (r0, r1, r2, r3)
        sid = lax.axis_index("s")
        w = sid * nc + lax.axis_index("c")
        s = base * w + jnp.minimum(w, extra)
        idx_start = s * _C

        @pl.when(sid == 0)
        def _():
            pltpu.sync_copy(table_hbm, tab_sp)
        plsc.subcore_barrier()

        @pl.when(w < extra)
        def _():
            pltpu.sync_copy(idx_hbm.at[pl.ds(idx_start, len_hi)],
                            idx_all.at[pl.ds(0, len_hi)])

        @pl.when(jnp.logical_and(w >= extra, w < nw - 1))
        def _():
            pltpu.sync_copy(idx_hbm.at[pl.ds(idx_start, len_lo)],
                            idx_all.at[pl.ds(0, len_lo)])

        @pl.when(w == nw - 1)
        def _():
            pltpu.sync_copy(idx_hbm.at[pl.ds(idx_start, len_last)],
                            idx_all.at[pl.ds(0, len_last)])

        def gather_async(c, b):
            return pltpu.async_copy(
                tab_sp.at[idx_all.at[pl.ds(c * _C, _C)]], rows[b],
                gsem.at[b])

        def wait_gather(c, b):
            pltpu.make_async_copy(
                tab_sp.at[idx_all.at[pl.ds(c * _C, _C)]], rows[b],
                gsem.at[b]).wait()

        def wait_store(b):
            pltpu.make_async_copy(rows[b], out_hbm.at[pl.ds(0, _C), :],
                                  ssem.at[b]).wait()

        for c in range(_DIST):
            gather_async(c, c % _NBUF)

        for c in range(base):
            b = c % _NBUF
            wait_gather(c, b)
            pltpu.async_copy(rows[b], out_hbm.at[pl.ds((s + c) * _C, _C), :],
                             ssem.at[b])
            c2 = c + _DIST
            b2 = c2 % _NBUF
            if c2 < base:
                if c2 >= _NBUF:
                    wait_store(b2)
                gather_async(c2, b2)
            elif c2 == base:
                @pl.when(w < extra)
                def _(c2=c2, b2=b2):
                    wait_store(b2)
                    gather_async(c2, b2)

        @pl.when(w < extra)
        def _():
            b = base % _NBUF
            wait_gather(base, b)
            pltpu.sync_copy(rows[b], out_hbm.at[pl.ds((s + base) * _C, _C), :])
            for bb in range(_NBUF):
                if bb != base % _NBUF:
                    wait_store(bb)

        @pl.when(w >= extra)
        def _():
            for bb in range(_NBUF):
                wait_store(bb)

        if tail:
            @pl.when(w == nw - 1)
            def _():
                pltpu.async_copy(
                    tab_sp.at[idx_all.at[pl.ds(base * _C, tail)]],
                    rows_t.at[pl.ds(0, tail), :], gsem.at[0]).wait()
                pltpu.sync_copy(rows_t.at[pl.ds(0, tail), :],
                                out_hbm.at[pl.ds(full * _C, tail), :])

    return gather_kernel


@functools.lru_cache(maxsize=None)
def _build_tc_embed(n, d, dtype_name, blk):
    dtype = jnp.dtype(dtype_name)

    def body(idx_ref, tab_ref, out_ref):
        idx = idx_ref[...]
        lanes = lax.broadcasted_iota(jnp.int32, (blk, 128), 1)
        oh = (idx == lanes).astype(dtype)
        out_ref[...] = jnp.dot(oh, tab_ref[...],
                               preferred_element_type=dtype)

    return pl.pallas_call(
        body,
        grid=(n // blk,),
        in_specs=[
            pl.BlockSpec((blk, 1), lambda i: (i, 0)),
            pl.BlockSpec((128, d), lambda i: (0, 0)),
        ],
        out_specs=pl.BlockSpec((blk, d), lambda i: (i, 0)),
        out_shape=jax.ShapeDtypeStruct((n, d), dtype),
    )


def kernel(atom_types, pos, table):
    idx = jnp.reshape(atom_types, (-1,))
    tab = table.astype(pos.dtype)
    n = idx.shape[0]
    v, d = tab.shape
    out_sc = _build_sc_gather(n, v, d, str(tab.dtype))(idx, tab)
    tab_pad = jnp.pad(tab, ((0, 128 - v), (0, 0)))
    out_tc = _build_tc_embed(n, d, str(tab.dtype), 2000)(atom_types, tab_pad)
    return (out_sc, out_tc)


# speedup vs baseline: 1.9546x; 1.9546x over previous
import functools

import jax
import jax.numpy as jnp
from jax import lax
from jax.experimental import pallas as pl
from jax.experimental.pallas import tpu as pltpu
from jax.experimental.pallas import tpu_sc as plsc

_C = 128
_NBUF = 4
_DIST = 2


@functools.lru_cache(maxsize=None)
def _build_sc_gather(n, v, d, dtype_name):
    dtype = jnp.dtype(dtype_name)
    info = plsc.get_sparse_core_info()
    nc, ns = info.num_cores, info.num_subcores
    nw = nc * ns
    full = n // _C
    tail = n % _C
    base = full // nw
    extra = full % nw
    assert base >= _NBUF and tail % 8 == 0 and extra < nw - 1

    len_lo = base * _C
    len_hi = (base + 1) * _C
    len_last = base * _C + tail

    mesh = plsc.VectorSubcoreMesh(core_axis_name="c", subcore_axis_name="s")

    scratch = [
        pltpu.VMEM((len_hi,), jnp.int32),
        pltpu.VMEM((_C, d), dtype),
        pltpu.VMEM((_C, d), dtype),
        pltpu.VMEM((_C, d), dtype),
        pltpu.VMEM((_C, d), dtype),
        pltpu.VMEM((max(tail, 8), d), dtype),
        pltpu.SemaphoreType.DMA((_NBUF,)),
        pltpu.SemaphoreType.DMA((_NBUF,)),
        pltpu.SemaphoreType.DMA((_NBUF,)),
        pltpu.VMEM_SHARED((v, d), dtype),
    ]

    out_t = jax.ShapeDtypeStruct((n, d), dtype)

    @functools.partial(
        pl.kernel,
        mesh=mesh,
        out_type=(out_t, out_t),
        scratch_types=scratch,
    )
    def gather_kernel(idx_hbm, table_hbm, out0_hbm, out1_hbm, idx_all,
                      r0, r1, r2, r3, rows_t, gsem, s0sem, s1sem, tab_sp):
        rows = (r0, r1, r2, r3)
        outs = (out0_hbm, out1_hbm)
        ssems = (s0sem, s1sem)
        sid = lax.axis_index("s")
        w = sid * nc + lax.axis_index("c")
        s = base * w + jnp.minimum(w, extra)
        idx_start = s * _C

        @pl.when(sid == 0)
        def _():
            pltpu.sync_copy(table_hbm, tab_sp)
        plsc.subcore_barrier()

        @pl.when(w < extra)
        def _():
            pltpu.sync_copy(idx_hbm.at[pl.ds(idx_start, len_hi)],
                            idx_all.at[pl.ds(0, len_hi)])

        @pl.when(jnp.logical_and(w >= extra, w < nw - 1))
        def _():
            pltpu.sync_copy(idx_hbm.at[pl.ds(idx_start, len_lo)],
                            idx_all.at[pl.ds(0, len_lo)])

        @pl.when(w == nw - 1)
        def _():
            pltpu.sync_copy(idx_hbm.at[pl.ds(idx_start, len_last)],
                            idx_all.at[pl.ds(0, len_last)])

        def gather_async(c, b):
            return pltpu.async_copy(
                tab_sp.at[idx_all.at[pl.ds(c * _C, _C)]], rows[b],
                gsem.at[b])

        def wait_gather(c, b):
            pltpu.make_async_copy(
                tab_sp.at[idx_all.at[pl.ds(c * _C, _C)]], rows[b],
                gsem.at[b]).wait()

        def wait_store(b):
            for o in range(2):
                pltpu.make_async_copy(rows[b], outs[o].at[pl.ds(0, _C), :],
                                      ssems[o].at[b]).wait()

        for c in range(_DIST):
            gather_async(c, c % _NBUF)

        for c in range(base):
            b = c % _NBUF
            wait_gather(c, b)
            for o in range(2):
                pltpu.async_copy(rows[b],
                                 outs[o].at[pl.ds((s + c) * _C, _C), :],
                                 ssems[o].at[b])
            c2 = c + _DIST
            b2 = c2 % _NBUF
            if c2 < base:
                if c2 >= _NBUF:
                    wait_store(b2)
                gather_async(c2, b2)
            elif c2 == base:
                @pl.when(w < extra)
                def _(c2=c2, b2=b2):
                    wait_store(b2)
                    gather_async(c2, b2)

        @pl.when(w < extra)
        def _():
            b = base % _NBUF
            wait_gather(base, b)
            for o in range(2):
                pltpu.sync_copy(rows[b],
                                outs[o].at[pl.ds((s + base) * _C, _C), :])
            for bb in range(_NBUF):
                if bb != base % _NBUF:
                    wait_store(bb)

        @pl.when(w >= extra)
        def _():
            for bb in range(_NBUF):
                wait_store(bb)

        if tail:
            @pl.when(w == nw - 1)
            def _():
                pltpu.async_copy(
                    tab_sp.at[idx_all.at[pl.ds(base * _C, tail)]],
                    rows_t.at[pl.ds(0, tail), :], gsem.at[0]).wait()
                for o in range(2):
                    pltpu.sync_copy(rows_t.at[pl.ds(0, tail), :],
                                    outs[o].at[pl.ds(full * _C, tail), :])

    return gather_kernel


def kernel(atom_types, pos, table):
    idx = jnp.reshape(atom_types, (-1,))
    tab = table.astype(pos.dtype)
    n = idx.shape[0]
    v, d = tab.shape
    out0, out1 = _build_sc_gather(n, v, d, str(tab.dtype))(idx, tab)
    return (out0, out1)
